# baseline (speedup 1.0000x reference)
import functools

import jax
import jax.numpy as jnp
from jax import lax
from jax.experimental import pallas as pl
from jax.experimental.pallas import tpu as pltpu
from jax.experimental.pallas import tpu_sc as plsc

VOCAB = 1000000
D = 64
CTX = 200

NC = 2
NS = 16
NW = NC * NS
ROWS_PER_W = 8
CTX_PAD = (NW + 1) * ROWS_PER_W
ACTIVE_W = CTX // ROWS_PER_W

BLK = 32768


def _sc_gather_mean(ctx_pad, in_emb):
    mesh = plsc.VectorSubcoreMesh(core_axis_name="c", subcore_axis_name="s")

    @functools.partial(
        pl.kernel,
        out_type=jax.ShapeDtypeStruct((NW, D), jnp.float32),
        mesh=mesh,
        scratch_types=[
            pltpu.VMEM((16,), jnp.int32),
            pltpu.VMEM((ROWS_PER_W, D), jnp.float32),
            pltpu.VMEM((D,), jnp.float32),
            pltpu.SemaphoreType.DMA,
        ],
        compiler_params=pltpu.CompilerParams(needs_layout_passes=False),
    )
    def k(ctx_hbm, emb_hbm, out_hbm, idx_v, rows_v, acc_v, sem):
        wid = lax.axis_index("s") * NC + lax.axis_index("c")
        base = wid * ROWS_PER_W
        pltpu.sync_copy(ctx_hbm.at[pl.ds(base, 16)], idx_v)
        idx_vec = idx_v[...]
        lanes = lax.iota(jnp.int32, 16)
        copies = []
        for j in range(ROWS_PER_W):
            ij = jnp.sum(idx_vec * (lanes == j).astype(jnp.int32))
            copies.append(
                pltpu.async_copy(
                    emb_hbm.at[pl.ds(ij, 1)], rows_v.at[pl.ds(j, 1)], sem
                )
            )
        for cp in copies:
            cp.wait()
        scale = jnp.where(wid < ACTIVE_W, jnp.float32(1.0 / CTX), jnp.float32(0.0))
        for c in range(D // 16):
            s = rows_v[0, pl.ds(c * 16, 16)]
            for i in range(1, ROWS_PER_W):
                s = s + rows_v[i, pl.ds(c * 16, 16)]
            acc_v[pl.ds(c * 16, 16)] = s * scale
        pltpu.sync_copy(acc_v, out_hbm.at[wid])

    return k(ctx_pad, in_emb)


def _tc_matvec(partials, out_emb):
    PAIRS = VOCAB // 2
    CH = 16384
    grid = pl.cdiv(PAIRS, CH)
    e2 = out_emb.reshape(PAIRS, 2 * D)

    def body(part_ref, emb_ref, out_ref):
        v = jnp.sum(part_ref[...], axis=0)
        s = jnp.sum(emb_ref[...]) + jnp.sum(v)
        out_ref[...] = jnp.broadcast_to(s, (1, 1, 2 * CH))

    out2 = pl.pallas_call(
        body,
        grid=(grid,),
        in_specs=[
            pl.BlockSpec((NW, D), lambda i: (0, 0)),
            pl.BlockSpec((CH, 2 * D), lambda i: (i, 0)),
        ],
        out_specs=pl.BlockSpec((1, 1, 2 * CH), lambda i: (i, 0, 0)),
        out_shape=jax.ShapeDtypeStruct((grid, 1, 2 * CH), jnp.float32),
    )(partials, e2)
    return out2.reshape(-1)[:VOCAB]


def kernel(context, in_emb, out_emb):
    ctx_pad = jnp.zeros((CTX_PAD,), jnp.int32).at[:CTX].set(context.astype(jnp.int32))
    partials = _sc_gather_mean(ctx_pad, in_emb)
    return _tc_matvec(partials, out_emb)


# speedup vs baseline: 1.0943x; 1.0943x over previous
import functools

import jax
import jax.numpy as jnp
from jax import lax
from jax.experimental import pallas as pl
from jax.experimental.pallas import tpu as pltpu
from jax.experimental.pallas import tpu_sc as plsc

VOCAB = 1000000
D = 64
CTX = 200

NC = 2
NS = 16
NW = NC * NS
ROWS_PER_W = 8
CTX_PAD = (NW + 1) * ROWS_PER_W
ACTIVE_W = CTX // ROWS_PER_W

BLK = 32768


def _sc_gather_mean(ctx_pad, in_emb):
    mesh = plsc.VectorSubcoreMesh(core_axis_name="c", subcore_axis_name="s")

    @functools.partial(
        pl.kernel,
        out_type=jax.ShapeDtypeStruct((NW, D), jnp.float32),
        mesh=mesh,
        scratch_types=[
            pltpu.VMEM((16,), jnp.int32),
            pltpu.VMEM((ROWS_PER_W, D), jnp.float32),
            pltpu.VMEM((D,), jnp.float32),
            pltpu.SemaphoreType.DMA,
        ],
        compiler_params=pltpu.CompilerParams(needs_layout_passes=False),
    )
    def k(ctx_hbm, emb_hbm, out_hbm, idx_v, rows_v, acc_v, sem):
        wid = lax.axis_index("s") * NC + lax.axis_index("c")
        base = wid * ROWS_PER_W
        pltpu.sync_copy(ctx_hbm.at[pl.ds(base, 16)], idx_v)
        idx_vec = idx_v[...]
        lanes = lax.iota(jnp.int32, 16)
        copies = []
        for j in range(ROWS_PER_W):
            ij = jnp.sum(idx_vec * (lanes == j).astype(jnp.int32))
            copies.append(
                pltpu.async_copy(
                    emb_hbm.at[pl.ds(ij, 1)], rows_v.at[pl.ds(j, 1)], sem
                )
            )
        for cp in copies:
            cp.wait()
        scale = jnp.where(wid < ACTIVE_W, jnp.float32(1.0 / CTX), jnp.float32(0.0))
        for c in range(D // 16):
            s = rows_v[0, pl.ds(c * 16, 16)]
            for i in range(1, ROWS_PER_W):
                s = s + rows_v[i, pl.ds(c * 16, 16)]
            acc_v[pl.ds(c * 16, 16)] = s * scale
        pltpu.sync_copy(acc_v, out_hbm.at[wid])

    return k(ctx_pad, in_emb)


NBUF = 5
CHUNK = 8000
NCH = VOCAB // CHUNK
NGRP = NCH // NBUF


def _tc_matvec(partials, out_emb):
    def body(part_ref, emb_hbm, out_hbm, *scratch):
        bufs = scratch[0:NBUF]
        outs = scratch[NBUF:2 * NBUF]
        isems = scratch[2 * NBUF:3 * NBUF]
        osems = scratch[3 * NBUF:4 * NBUF]
        v = jnp.sum(part_ref[...], axis=0).reshape(1, D)

        def in_cp(c, b):
            return pltpu.make_async_copy(
                emb_hbm.at[pl.ds(c * CHUNK, CHUNK), :], bufs[b], isems[b])

        def out_cp(c, b):
            return pltpu.make_async_copy(outs[b], out_hbm.at[c], osems[b])

        for b in range(NBUF):
            in_cp(b, b).start()

        def grp(g, carry):
            for b in range(NBUF):
                c = g * NBUF + b
                in_cp(c, b).wait()
                et = bufs[b][...].T
                s = jax.lax.dot_general(
                    v, et, (((1,), (0,)), ((), ())),
                    preferred_element_type=jnp.float32)

                @pl.when(g > 0)
                def _():
                    out_cp(c - NBUF, b).wait()

                outs[b][...] = s
                out_cp(c, b).start()

                @pl.when(c + NBUF < NCH)
                def _():
                    in_cp(c + NBUF, b).start()
            return carry

        lax.fori_loop(0, NGRP, grp, 0)
        for b in range(NBUF):
            out_cp(NCH - NBUF + b, b).wait()

    out2 = pl.pallas_call(
        body,
        in_specs=[
            pl.BlockSpec(memory_space=pltpu.MemorySpace.VMEM),
            pl.BlockSpec(memory_space=pltpu.MemorySpace.HBM),
        ],
        out_specs=pl.BlockSpec(memory_space=pltpu.MemorySpace.HBM),
        out_shape=jax.ShapeDtypeStruct((NCH, 1, CHUNK), jnp.float32),
        scratch_shapes=(
            [pltpu.VMEM((CHUNK, D), jnp.float32) for _ in range(NBUF)]
            + [pltpu.VMEM((1, CHUNK), jnp.float32) for _ in range(NBUF)]
            + [pltpu.SemaphoreType.DMA for _ in range(2 * NBUF)]
        ),
    )(partials, out_emb)
    return out2.reshape(-1)


def kernel(context, in_emb, out_emb):
    ctx_pad = jnp.zeros((CTX_PAD,), jnp.int32).at[:CTX].set(context.astype(jnp.int32))
    partials = _sc_gather_mean(ctx_pad, in_emb)
    return _tc_matvec(partials, out_emb)


# speedup vs baseline: 1.4156x; 1.2937x over previous
import functools

import jax
import jax.numpy as jnp
from jax import lax
from jax.experimental import pallas as pl
from jax.experimental.pallas import tpu as pltpu
from jax.experimental.pallas import tpu_sc as plsc

VOCAB = 1000000
D = 64
CTX = 200

NC = 2
NS = 16
NW = NC * NS
ROWS_PER_W = 8
CTX_PAD = (NW + 1) * ROWS_PER_W
ACTIVE_W = CTX // ROWS_PER_W

BLK = 32768


def _sc_gather_mean(ctx_pad, in_emb):
    mesh = plsc.VectorSubcoreMesh(core_axis_name="c", subcore_axis_name="s")

    @functools.partial(
        pl.kernel,
        out_type=jax.ShapeDtypeStruct((NW, D), jnp.float32),
        mesh=mesh,
        scratch_types=[
            pltpu.VMEM((16,), jnp.int32),
            pltpu.VMEM((ROWS_PER_W, D), jnp.float32),
            pltpu.VMEM((D,), jnp.float32),
            pltpu.SemaphoreType.DMA,
        ],
        compiler_params=pltpu.CompilerParams(needs_layout_passes=False),
    )
    def k(ctx_hbm, emb_hbm, out_hbm, idx_v, rows_v, acc_v, sem):
        wid = lax.axis_index("s") * NC + lax.axis_index("c")
        base = wid * ROWS_PER_W
        pltpu.sync_copy(ctx_hbm.at[pl.ds(base, 16)], idx_v)
        idx_vec = idx_v[...]
        lanes = lax.iota(jnp.int32, 16)
        copies = []
        for j in range(ROWS_PER_W):
            ij = jnp.sum(idx_vec * (lanes == j).astype(jnp.int32))
            copies.append(
                pltpu.async_copy(
                    emb_hbm.at[pl.ds(ij, 1)], rows_v.at[pl.ds(j, 1)], sem
                )
            )
        for cp in copies:
            cp.wait()
        scale = jnp.where(wid < ACTIVE_W, jnp.float32(1.0 / CTX), jnp.float32(0.0))
        for c in range(D // 16):
            s = rows_v[0, pl.ds(c * 16, 16)]
            for i in range(1, ROWS_PER_W):
                s = s + rows_v[i, pl.ds(c * 16, 16)]
            acc_v[pl.ds(c * 16, 16)] = s * scale
        pltpu.sync_copy(acc_v, out_hbm.at[wid])

    return k(ctx_pad, in_emb)


NBUF = 5
CHUNK = 8000
NCH = VOCAB // CHUNK
NGRP = NCH // NBUF


def _tc_matvec(partials, out_emb):
    def body(part_ref, emb_hbm, out_hbm, *scratch):
        bufs = scratch[0:NBUF]
        outs = scratch[NBUF:2 * NBUF]
        isems = scratch[2 * NBUF:3 * NBUF]
        osems = scratch[3 * NBUF:4 * NBUF]
        v = jnp.sum(part_ref[...], axis=0).reshape(1, D)

        def in_cp(c, b):
            return pltpu.make_async_copy(
                emb_hbm.at[pl.ds(c * CHUNK, CHUNK), :], bufs[b], isems[b])

        def out_cp(c, b):
            return pltpu.make_async_copy(outs[b], out_hbm.at[c], osems[b])

        for b in range(NBUF):
            in_cp(b, b).start()

        def grp(g, carry):
            for b in range(NBUF):
                c = g * NBUF + b
                in_cp(c, b).wait()
                et = bufs[b][...].T
                s = jax.lax.dot_general(
                    v, et, (((1,), (0,)), ((), ())),
                    preferred_element_type=jnp.float32)

                @pl.when(g > 0)
                def _():
                    out_cp(c - NBUF, b).wait()

                outs[b][...] = s
                out_cp(c, b).start()

                @pl.when(c + NBUF < NCH)
                def _():
                    in_cp(c + NBUF, b).start()
            return carry

        lax.fori_loop(0, NGRP, grp, 0)
        for b in range(NBUF):
            out_cp(NCH - NBUF + b, b).wait()

    out2 = pl.pallas_call(
        body,
        in_specs=[
            pl.BlockSpec(memory_space=pltpu.MemorySpace.VMEM),
            pl.BlockSpec(memory_space=pltpu.MemorySpace.HBM),
        ],
        out_specs=pl.BlockSpec(memory_space=pltpu.MemorySpace.HBM),
        out_shape=jax.ShapeDtypeStruct((NCH, 1, CHUNK), jnp.float32),
        scratch_shapes=(
            [pltpu.VMEM((CHUNK, D), jnp.float32) for _ in range(NBUF)]
            + [pltpu.VMEM((1, CHUNK), jnp.float32) for _ in range(NBUF)]
            + [pltpu.SemaphoreType.DMA for _ in range(2 * NBUF)]
        ),
    )(partials, out_emb)
    return out2.reshape(-1)


def kernel(context, in_emb, out_emb):
    v = jnp.take(in_emb, context, axis=0).mean(axis=0)
    partials = jnp.tile((v / NW)[None, :], (NW, 1))
    return _tc_matvec(partials, out_emb)
